# baseline (speedup 1.0000x reference)
import functools

import jax
import jax.numpy as jnp
from jax import lax
from jax.experimental import pallas as pl
from jax.experimental.pallas import tpu as pltpu
from jax.experimental.pallas import tpu_sc as plsc

DIM = 33
NUM = 10
NPIX = 512 * 512
NBIN = 4096
LANES = 16
NCORE = 2
NSUB = 16
NTILE = NCORE * NSUB
LUTLEN = 3 * DIM * DIM * DIM
LUTPAD = LUTLEN + (-LUTLEN) % 16
CHUNK = 2048
PPT1 = NPIX // NSUB
PPT3 = NPIX // NTILE

_mesh = plsc.VectorSubcoreMesh(core_axis_name="c", subcore_axis_name="s")


HWORDS = NBIN * LANES
BSLICE = NBIN // NSUB
SUBW = NBIN * LANES


@functools.partial(
    pl.kernel,
    mesh=_mesh,
    out_type=jax.ShapeDtypeStruct((NTILE * LANES,), jnp.float32),
    compiler_params=pltpu.CompilerParams(needs_layout_passes=False),
    scratch_types=[
        pltpu.VMEM((HWORDS,), jnp.float32),
        pltpu.VMEM((NBIN,), jnp.float32),
        pltpu.VMEM_SHARED((NSUB * NBIN,), jnp.float32),
        pltpu.VMEM((32, 512), jnp.float32),
        pltpu.VMEM((32, 512), jnp.float32),
        pltpu.VMEM((32, 512), jnp.float32),
        pltpu.VMEM((BSLICE,), jnp.float32),
        pltpu.VMEM((BSLICE,), jnp.float32),
        pltpu.VMEM((BSLICE * LANES,), jnp.float32),
        pltpu.VMEM((LANES,), jnp.float32),
    ],
)
def _sc_feat(img_m, img_l, subs, zer, part, hist, comp, shared,
             rB, gB, bB, hacc, htmp, sbuf, accbuf):
    c = lax.axis_index("c")
    s = lax.axis_index("s")
    lane = lax.iota(jnp.int32, LANES)
    lane_major = lane * NBIN
    ones = jnp.full((LANES,), 1.0, jnp.float32)

    pltpu.sync_copy(zer, hist)

    for k in range(2):
        row0 = s * 32 + k * 16
        pltpu.sync_copy(img_m.at[0, 0, pl.ds(row0, 16), :], rB.at[pl.ds(0, 16), :])
        pltpu.sync_copy(img_m.at[0, 1, pl.ds(row0, 16), :], gB.at[pl.ds(0, 16), :])
        pltpu.sync_copy(img_m.at[0, 2, pl.ds(row0, 16), :], bB.at[pl.ds(0, 16), :])
        pltpu.sync_copy(img_l.at[0, 0, pl.ds(row0, 16), :], rB.at[pl.ds(16, 16), :])
        pltpu.sync_copy(img_l.at[0, 1, pl.ds(row0, 16), :], gB.at[pl.ds(16, 16), :])
        pltpu.sync_copy(img_l.at[0, 2, pl.ds(row0, 16), :], bB.at[pl.ds(16, 16), :])

        for i in range(16):
            def hbody(j, _):
                sl = pl.ds(j * LANES, LANES)
                row = c * 16 + i
                r = rB[row, sl]
                g = gB[row, sl]
                b = bB[row, sl]
                p = (r * 4096.0 + g * 256.0 + b * 16.0).astype(jnp.int32) + lane
                plsc.addupdate_scatter(hist, [p], ones)
                return 0
            lax.fori_loop(0, 512 // LANES, hbody, 0, unroll=4)

    lane16 = lane * LANES

    def fbody(i, _):
        bin_base = i * LANES * LANES
        acc = plsc.load_gather(hist, [lane16 + bin_base])
        for l in range(1, LANES):
            acc = acc + plsc.load_gather(hist, [lane16 + (bin_base + l)])
        comp[pl.ds(i * LANES, LANES)] = acc
        return 0
    lax.fori_loop(0, NBIN // LANES, fbody, 0, unroll=2)

    pltpu.sync_copy(comp, shared.at[pl.ds(s * NBIN, NBIN)])
    plsc.subcore_barrier()

    pltpu.sync_copy(shared.at[pl.ds(s * BSLICE, BSLICE)], hacc)

    def mbody(t, _):
        pltpu.sync_copy(shared.at[pl.ds(t * NBIN + s * BSLICE, BSLICE)], htmp)
        for i in range(BSLICE // LANES):
            sl = pl.ds(i * LANES, LANES)
            hacc[sl] = hacc[sl] + htmp[sl]
        return 0
    lax.fori_loop(1, NSUB, mbody, 0)

    pltpu.sync_copy(subs.at[pl.ds(c * SUBW + s * BSLICE * LANES, BSLICE * LANES)],
                    sbuf)

    izero = lane * 0

    def rbody(i, acc):
        cnt = plsc.load_gather(hacc, [izero + i])
        return acc + cnt * sbuf[pl.ds(i * LANES, LANES)]
    acc = lax.fori_loop(0, BSLICE, rbody, jnp.zeros((LANES,), jnp.float32),
                        unroll=4)
    accbuf[:] = acc
    pltpu.sync_copy(accbuf, part.at[pl.ds((c * NSUB + s) * LANES, LANES)])


def _tc_body(pq_ref, lutcat_ref, sl_ref, wl_ref, l0_ref, l1_ref, l2_ref,
             pre0_ref, pre1_ref, pre2_ref):
    fsum = jnp.sum(pq_ref[:], axis=0)
    feat = fsum * (1.0 / NPIX)
    featq = jnp.clip(jnp.round(feat * 2.0) / 2.0, -16.0, 15.5)
    mid = (featq * 2.0).astype(jnp.int32) + 32
    mid0 = mid[:, 0:1]
    mid1 = mid[:, 1:2]
    jj = lax.broadcasted_iota(jnp.int32, (5, 4096), 1)
    onehot = ((jj >> 6) == mid0) & ((jj & 63) == mid1)
    of = onehot.astype(jnp.float32)
    oc = jnp.sum(of[:, :, None] * lutcat_ref[:], axis=1)
    weights = jnp.sum(oc, axis=0) / 4.0 - 40.0
    w3 = weights[None, :, None]
    for lr, prer in ((l0_ref, pre0_ref), (l1_ref, pre1_ref), (l2_ref, pre2_ref)):
        lw = jnp.sum(w3 * lr[:], axis=1)
        gw = jnp.dot(lw, wl_ref[:], preferred_element_type=jnp.float32)
        prer[:] = jnp.dot(sl_ref[:], gw, preferred_element_type=jnp.float32)


_OFFS = (0, 1, DIM, DIM + 1, DIM * DIM, DIM * DIM + 1, DIM * DIM + DIM,
         DIM * DIM + DIM + 1)
_POFFS = (0, DIM, DIM * DIM, DIM * DIM + DIM)
_INV = 32.0 / 1.000001


@functools.partial(
    pl.kernel,
    mesh=_mesh,
    out_type=jax.ShapeDtypeStruct((1, 3, 512, 512), jnp.float32),
    compiler_params=pltpu.CompilerParams(needs_layout_passes=False),
    scratch_types=[
        pltpu.VMEM((LUTPAD,), jnp.int32),
        pltpu.VMEM((8, 512), jnp.float32),
        pltpu.VMEM((8, 512), jnp.float32),
        pltpu.VMEM((8, 512), jnp.float32),
    ],
)
def _sc_interp(lutflat, org, out, lut_v, rb, gb, bb):
    c = lax.axis_index("c")
    s = lax.axis_index("s")
    wid = c * NSUB + s
    pltpu.sync_copy(lutflat, lut_v)
    base_row = wid * 16
    for k in range(2):
        r0 = base_row + k * 8
        pltpu.sync_copy(org.at[0, 0, pl.ds(r0, 8), :], rb)
        pltpu.sync_copy(org.at[0, 1, pl.ds(r0, 8), :], gb)
        pltpu.sync_copy(org.at[0, 2, pl.ds(r0, 8), :], bb)

        for i in range(8):
            @plsc.parallel_loop(0, 512 // LANES, unroll=2)
            def body(j):
                sl = pl.ds(j * LANES, LANES)
                r = rb[i, sl]
                g = gb[i, sl]
                b = bb[i, sl]
                x = r * _INV
                y = g * _INV
                z = b * _INV
                ri = jnp.minimum(x.astype(jnp.int32), DIM - 2)
                gi = jnp.minimum(y.astype(jnp.int32), DIM - 2)
                bi = jnp.minimum(z.astype(jnp.int32), DIM - 2)
                rd = x - ri.astype(jnp.float32)
                gd = y - gi.astype(jnp.float32)
                bd = z - bi.astype(jnp.float32)
                rm = 1.0 - rd
                gm = 1.0 - gd
                bm = 1.0 - bd
                w00 = rm * gm
                w10 = rd * gm
                w01 = rm * gd
                w11 = rd * gd
                ws = (w00 * bm, w10 * bm, w01 * bm, w11 * bm,
                      w00 * bd, w10 * bd, w01 * bd, w11 * bd)
                ib = (bi * DIM + gi) * DIM + ri
                vals = []
                himask = jnp.full((LANES,), -65536, jnp.int32)
                for ch in range(3):
                    o = ib + ch * (DIM * DIM * DIM)
                    v0 = None
                    v1 = None
                    for t in range(4):
                        gp = plsc.load_gather(lut_v, [o + _POFFS[t]])
                        lo = plsc.bitcast(gp << 16, jnp.float32)
                        hi = plsc.bitcast(gp & himask, jnp.float32)
                        v0 = ws[2 * t] * lo if v0 is None else v0 + ws[2 * t] * lo
                        v1 = ws[2 * t + 1] * hi if v1 is None else v1 + ws[2 * t + 1] * hi
                    vals.append(v0 + v1)
                rb[i, sl] = vals[0] + r
                gb[i, sl] = vals[1] + g
                bb[i, sl] = vals[2] + b
        pltpu.sync_copy(rb, out.at[0, 0, pl.ds(r0, 8), :])
        pltpu.sync_copy(gb, out.at[0, 1, pl.ds(r0, 8), :])
        pltpu.sync_copy(bb, out.at[0, 2, pl.ds(r0, 8), :])


def kernel(img_msb, img_lsb, img_org, feature_msb, feature_lsb, lut_cat,
           s_Layers, w_Layers, LUTs):
    img_m = img_msb
    img_l = img_lsb
    sub_m = feature_msb[::16, :]
    sub_l = feature_lsb[::16, :]
    subs = jnp.stack([jnp.pad(sub_m, ((0, 0), (0, LANES - NUM))),
                      jnp.pad(sub_l, ((0, 0), (0, LANES - NUM)))]).reshape(-1)
    zer = jnp.zeros((HWORDS,), jnp.float32)

    part = _sc_feat(img_m, img_l, subs, zer)
    pq = part.reshape(NTILE, 8, 2)[:, :5, :]

    l4 = LUTs.reshape(5, NUM, 3, NUM)
    pre0, pre1, pre2 = pl.pallas_call(
        _tc_body,
        out_shape=[jax.ShapeDtypeStruct((DIM, DIM * DIM), jnp.float32)] * 3,
    )(pq, lut_cat, s_Layers, w_Layers,
      l4[:, :, 0, :], l4[:, :, 1, :], l4[:, :, 2, :])

    p0 = pre0.reshape(DIM, DIM, DIM)
    p1 = pre1.reshape(DIM, DIM, DIM)
    p2 = pre2.reshape(DIM, DIM, DIM)
    lut0 = jnp.transpose(p0, (2, 0, 1))
    lut1 = jnp.transpose(p1, (0, 2, 1))
    d3lut = jnp.stack([lut0, lut1, p2])[None]

    lut3 = d3lut[0]
    hi3 = jnp.roll(lut3, -1, axis=-1)
    lo_u = lax.bitcast_convert_type(lut3.astype(jnp.bfloat16),
                                    jnp.uint16).astype(jnp.uint32)
    hi_u = lax.bitcast_convert_type(hi3.astype(jnp.bfloat16),
                                    jnp.uint16).astype(jnp.uint32)
    packed = lax.bitcast_convert_type(lo_u | (hi_u << 16), jnp.int32)
    lutflat = jnp.pad(packed.reshape(-1), (0, LUTPAD - LUTLEN))
    res = _sc_interp(lutflat, img_org)
    return (res, d3lut)


# speedup vs baseline: 1.0399x; 1.0399x over previous
import functools

import jax
import jax.numpy as jnp
from jax import lax
from jax.experimental import pallas as pl
from jax.experimental.pallas import tpu as pltpu
from jax.experimental.pallas import tpu_sc as plsc

DIM = 33
NUM = 10
NPIX = 512 * 512
NBIN = 4096
LANES = 16
NCORE = 2
NSUB = 16
NTILE = NCORE * NSUB
LUTLEN = 3 * DIM * DIM * DIM
LUTPAD = LUTLEN + (-LUTLEN) % 16
CHUNK = 2048
PPT1 = NPIX // NSUB
PPT3 = NPIX // NTILE

_mesh = plsc.VectorSubcoreMesh(core_axis_name="c", subcore_axis_name="s")


HWORDS = NBIN * LANES
BSLICE = NBIN // NSUB
SUBW = NBIN * LANES


@functools.partial(
    pl.kernel,
    mesh=_mesh,
    out_type=jax.ShapeDtypeStruct((NTILE * LANES,), jnp.float32),
    compiler_params=pltpu.CompilerParams(needs_layout_passes=False),
    scratch_types=[
        pltpu.VMEM((HWORDS,), jnp.float32),
        pltpu.VMEM((NBIN,), jnp.float32),
        pltpu.VMEM_SHARED((NSUB * NBIN,), jnp.float32),
        pltpu.VMEM((16, 512), jnp.float32),
        pltpu.VMEM((16, 512), jnp.float32),
        pltpu.VMEM((16, 512), jnp.float32),
        pltpu.VMEM((16, 512), jnp.float32),
        pltpu.VMEM((16, 512), jnp.float32),
        pltpu.VMEM((16, 512), jnp.float32),
        pltpu.VMEM((BSLICE,), jnp.float32),
        pltpu.VMEM((BSLICE,), jnp.float32),
        pltpu.VMEM((BSLICE * LANES,), jnp.float32),
        pltpu.VMEM((LANES,), jnp.float32),
    ],
)
def _sc_feat(img_m, img_l, subs, part, hist, comp, shared,
             rm, gm, bm, rl, gl, bl, hacc, htmp, sbuf, accbuf):
    c = lax.axis_index("c")
    s = lax.axis_index("s")
    lane = lax.iota(jnp.int32, LANES)
    ones = jnp.full((LANES,), 1.0, jnp.float32)
    zeros = jnp.zeros((LANES,), jnp.float32)

    def zbody(i, _):
        hist[pl.ds(i * LANES, LANES)] = zeros
        return 0
    lax.fori_loop(0, NBIN, zbody, 0, unroll=8)

    is_msb = c == 0
    for k in range(2):
        row0 = s * 32 + k * 16
        pltpu.sync_copy(img_m.at[0, 0, pl.ds(row0, 16), :], rm)
        pltpu.sync_copy(img_m.at[0, 1, pl.ds(row0, 16), :], gm)
        pltpu.sync_copy(img_m.at[0, 2, pl.ds(row0, 16), :], bm)
        pltpu.sync_copy(img_l.at[0, 0, pl.ds(row0, 16), :], rl)
        pltpu.sync_copy(img_l.at[0, 1, pl.ds(row0, 16), :], gl)
        pltpu.sync_copy(img_l.at[0, 2, pl.ds(row0, 16), :], bl)

        for i in range(16):
            def hbody(j, _):
                sl = pl.ds(j * LANES, LANES)
                r = jnp.where(is_msb, rm[i, sl], rl[i, sl])
                g = jnp.where(is_msb, gm[i, sl], gl[i, sl])
                b = jnp.where(is_msb, bm[i, sl], bl[i, sl])
                p = (r * 4096.0 + g * 256.0 + b * 16.0).astype(jnp.int32) + lane
                plsc.addupdate_scatter(hist, [p], ones)
                return 0
            lax.fori_loop(0, 512 // LANES, hbody, 0, unroll=4)

    lane16 = lane * LANES

    def fbody(i, _):
        bin_base = i * LANES * LANES
        acc = plsc.load_gather(hist, [lane16 + bin_base])
        for l in range(1, LANES):
            acc = acc + plsc.load_gather(hist, [lane16 + (bin_base + l)])
        comp[pl.ds(i * LANES, LANES)] = acc
        return 0
    lax.fori_loop(0, NBIN // LANES, fbody, 0, unroll=2)

    pltpu.sync_copy(comp, shared.at[pl.ds(s * NBIN, NBIN)])
    plsc.subcore_barrier()

    pltpu.sync_copy(shared.at[pl.ds(s * BSLICE, BSLICE)], hacc)

    def mbody(t, _):
        pltpu.sync_copy(shared.at[pl.ds(t * NBIN + s * BSLICE, BSLICE)], htmp)
        for i in range(BSLICE // LANES):
            sl = pl.ds(i * LANES, LANES)
            hacc[sl] = hacc[sl] + htmp[sl]
        return 0
    lax.fori_loop(1, NSUB, mbody, 0)

    pltpu.sync_copy(subs.at[pl.ds(c * SUBW + s * BSLICE * LANES, BSLICE * LANES)],
                    sbuf)

    izero = lane * 0

    def rbody(i, acc):
        cnt = plsc.load_gather(hacc, [izero + i])
        return acc + cnt * sbuf[pl.ds(i * LANES, LANES)]
    acc = lax.fori_loop(0, BSLICE, rbody, jnp.zeros((LANES,), jnp.float32),
                        unroll=4)
    accbuf[:] = acc
    pltpu.sync_copy(accbuf, part.at[pl.ds((c * NSUB + s) * LANES, LANES)])


def _tc_body(pq_ref, lutcat_ref, sl_ref, wl_ref, l0_ref, l1_ref, l2_ref,
             pre0_ref, pre1_ref, pre2_ref):
    fsum = jnp.sum(pq_ref[:], axis=0)
    feat = fsum * (1.0 / NPIX)
    featq = jnp.clip(jnp.round(feat * 2.0) / 2.0, -16.0, 15.5)
    mid = (featq * 2.0).astype(jnp.int32) + 32
    mid0 = mid[:, 0:1]
    mid1 = mid[:, 1:2]
    jj = lax.broadcasted_iota(jnp.int32, (5, 4096), 1)
    onehot = ((jj >> 6) == mid0) & ((jj & 63) == mid1)
    of = onehot.astype(jnp.float32)
    oc = jnp.sum(of[:, :, None] * lutcat_ref[:], axis=1)
    weights = jnp.sum(oc, axis=0) / 4.0 - 40.0
    w3 = weights[None, :, None]
    for lr, prer in ((l0_ref, pre0_ref), (l1_ref, pre1_ref), (l2_ref, pre2_ref)):
        lw = jnp.sum(w3 * lr[:], axis=1)
        gw = jnp.dot(lw, wl_ref[:], preferred_element_type=jnp.float32)
        prer[:] = jnp.dot(sl_ref[:], gw, preferred_element_type=jnp.float32)


_OFFS = (0, 1, DIM, DIM + 1, DIM * DIM, DIM * DIM + 1, DIM * DIM + DIM,
         DIM * DIM + DIM + 1)
_POFFS = (0, DIM, DIM * DIM, DIM * DIM + DIM)
_INV = 32.0 / 1.000001


@functools.partial(
    pl.kernel,
    mesh=_mesh,
    out_type=jax.ShapeDtypeStruct((1, 3, 512, 512), jnp.float32),
    compiler_params=pltpu.CompilerParams(needs_layout_passes=False),
    scratch_types=[
        pltpu.VMEM((LUTPAD,), jnp.float32),
        pltpu.VMEM((8, 512), jnp.float32),
        pltpu.VMEM((8, 512), jnp.float32),
        pltpu.VMEM((8, 512), jnp.float32),
    ],
)
def _sc_interp(lutflat, org, out, lut_v, rb, gb, bb):
    c = lax.axis_index("c")
    s = lax.axis_index("s")
    wid = c * NSUB + s
    pltpu.sync_copy(lutflat, lut_v)
    base_row = wid * 16
    for k in range(2):
        r0 = base_row + k * 8
        pltpu.sync_copy(org.at[0, 0, pl.ds(r0, 8), :], rb)
        pltpu.sync_copy(org.at[0, 1, pl.ds(r0, 8), :], gb)
        pltpu.sync_copy(org.at[0, 2, pl.ds(r0, 8), :], bb)

        for i in range(8):
            def body(j, _):
                sl = pl.ds(j * LANES, LANES)
                r = rb[i, sl]
                g = gb[i, sl]
                b = bb[i, sl]
                x = r * _INV
                y = g * _INV
                z = b * _INV
                ri = jnp.minimum(x.astype(jnp.int32), DIM - 2)
                gi = jnp.minimum(y.astype(jnp.int32), DIM - 2)
                bi = jnp.minimum(z.astype(jnp.int32), DIM - 2)
                rd = x - ri.astype(jnp.float32)
                gd = y - gi.astype(jnp.float32)
                bd = z - bi.astype(jnp.float32)
                rm = 1.0 - rd
                gm = 1.0 - gd
                bm = 1.0 - bd
                w00 = rm * gm
                w10 = rd * gm
                w01 = rm * gd
                w11 = rd * gd
                ws = (w00 * bm, w10 * bm, w01 * bm, w11 * bm,
                      w00 * bd, w10 * bd, w01 * bd, w11 * bd)
                ib = (bi * DIM + gi) * DIM + ri
                vals = []
                for ch in range(3):
                    o = ib + ch * (DIM * DIM * DIM)
                    v = ws[0] * plsc.load_gather(lut_v, [o])
                    for t in range(1, 8):
                        v = v + ws[t] * plsc.load_gather(lut_v, [o + _OFFS[t]])
                    vals.append(v)
                rb[i, sl] = vals[0] + r
                gb[i, sl] = vals[1] + g
                bb[i, sl] = vals[2] + b
                return 0
            lax.fori_loop(0, 512 // LANES, body, 0)
        pltpu.sync_copy(rb, out.at[0, 0, pl.ds(r0, 8), :])
        pltpu.sync_copy(gb, out.at[0, 1, pl.ds(r0, 8), :])
        pltpu.sync_copy(bb, out.at[0, 2, pl.ds(r0, 8), :])


def kernel(img_msb, img_lsb, img_org, feature_msb, feature_lsb, lut_cat,
           s_Layers, w_Layers, LUTs):
    img_m = img_msb
    img_l = img_lsb
    sub_m = feature_msb[::16, :]
    sub_l = feature_lsb[::16, :]
    subs = jnp.stack([jnp.pad(sub_m, ((0, 0), (0, LANES - NUM))),
                      jnp.pad(sub_l, ((0, 0), (0, LANES - NUM)))]).reshape(-1)

    part = _sc_feat(img_m, img_l, subs)
    pq = part.reshape(NTILE, 8, 2)[:, :5, :]

    l4 = LUTs.reshape(5, NUM, 3, NUM)
    pre0, pre1, pre2 = pl.pallas_call(
        _tc_body,
        out_shape=[jax.ShapeDtypeStruct((DIM, DIM * DIM), jnp.float32)] * 3,
    )(pq, lut_cat, s_Layers, w_Layers,
      l4[:, :, 0, :], l4[:, :, 1, :], l4[:, :, 2, :])

    p0 = pre0.reshape(DIM, DIM, DIM)
    p1 = pre1.reshape(DIM, DIM, DIM)
    p2 = pre2.reshape(DIM, DIM, DIM)
    lut0 = jnp.transpose(p0, (2, 0, 1))
    lut1 = jnp.transpose(p1, (0, 2, 1))
    d3lut = jnp.stack([lut0, lut1, p2])[None]

    lutflat = jnp.pad(d3lut.reshape(-1), (0, LUTPAD - LUTLEN))
    res = _sc_interp(lutflat, img_org)
    return (res, d3lut)
